# Initial kernel scaffold; baseline (speedup 1.0000x reference)
import jax
import jax.numpy as jnp
from jax.experimental import pallas as pl


def kernel(g, z, pair_nodes, z_table, bias0, bias1, bias2, W1, b1, W2, b2):
    raise NotImplementedError("write your pallas kernel here")


# speedup vs baseline: 7.1886x; 7.1886x over previous
import functools

import jax
import jax.numpy as jnp
from jax import lax
from jax.experimental import pallas as pl
from jax.experimental.pallas import tpu as pltpu
from jax.experimental.pallas import tpu_sc as plsc

N = 10000
NP = 10240
E = 320000
H = 128
NC = 2
NS = 16
NW = NC * NS
NH = NP // NC
AGG_ROWS = NH + 16
DUMMY = NH
NT = NH // NS
EPW = E // NW
EPT = E // NS
SCAN = 2000
CH = 128
CAP = EPT + 96

_MESH = plsc.VectorSubcoreMesh(core_axis_name="c", subcore_axis_name="s",
                               num_cores=NC, num_subcores=NS)
_PARAMS = pltpu.CompilerParams(needs_layout_passes=False)

def _ones_f():
    return jnp.ones((16,), jnp.float32)


def _zero_i():
    return jnp.zeros((16,), jnp.int32)


def _mask_to_i32(m):
    return jnp.where(m, jnp.ones((16,), jnp.int32), _zero_i())


def _deg_body(src_hbm, dst_hbm, pair_hbm,
              dego_hbm, degi_hbm, flg_hbm,
              sbuf, dbuf, dego, degi, flg, pairb):
    c = lax.axis_index("c")
    s = lax.axis_index("s")
    wid = c * NS + s

    def zero(j, _):
        z = jnp.zeros((16,), jnp.float32)
        dego[pl.ds(j * 16, 16)] = z
        degi[pl.ds(j * 16, 16)] = z
        flg[pl.ds(j * 16, 16)] = z
        return _

    lax.fori_loop(0, NP // 16, zero, 0)
    pltpu.sync_copy(pair_hbm, pairb)
    pairv = pairb[pl.ds(0, 16)]
    u = pairv[0]
    v = pairv[1]

    ebase = wid * EPW
    for k in range(EPW // SCAN):
        pltpu.sync_copy(src_hbm.at[pl.ds(ebase + k * SCAN, SCAN)], sbuf)
        pltpu.sync_copy(dst_hbm.at[pl.ds(ebase + k * SCAN, SCAN)], dbuf)

        def grp(g, _):
            sv = sbuf[pl.ds(g * 16, 16)]
            dv = dbuf[pl.ds(g * 16, 16)]
            plsc.addupdate_scatter(dego, [sv], _ones_f())
            plsc.addupdate_scatter(degi, [dv], _ones_f())
            mp = (dv == u) | (dv == v)
            plsc.addupdate_scatter(flg, [sv], _ones_f(), mask=mp)
            return _

        lax.fori_loop(0, SCAN // 16, grp, 0)

    pltpu.sync_copy(dego, dego_hbm.at[wid])
    pltpu.sync_copy(degi, degi_hbm.at[wid])
    pltpu.sync_copy(flg, flg_hbm.at[wid])


def _k1(src, dst, pair16):
    out = jax.ShapeDtypeStruct((NW, NP), jnp.float32)
    return pl.kernel(
        _deg_body,
        out_type=(out, out, out),
        mesh=_MESH,
        compiler_params=_PARAMS,
        scratch_types=[
            pltpu.VMEM((SCAN,), jnp.int32),
            pltpu.VMEM((SCAN,), jnp.int32),
            pltpu.VMEM((NP,), jnp.float32),
            pltpu.VMEM((NP,), jnp.float32),
            pltpu.VMEM((NP,), jnp.float32),
            pltpu.VMEM((16,), jnp.int32),
        ],
    )(src, dst, pair16)


def _norm_body(po, pi, pf, no_ref, ni_ref, fl_ref):
    so = jnp.sum(po[...], axis=0)
    si = jnp.sum(pi[...], axis=0)
    sf = jnp.sum(pf[...], axis=0)
    no_ref[...] = jnp.power(jnp.maximum(so, 1.0), -0.5)
    ni_ref[...] = jnp.power(jnp.maximum(si, 1.0), -0.5)
    fl_ref[...] = jnp.where(sf > 0.0, 1.0, 0.0)


def _k2(po, pi, pf):
    o = jax.ShapeDtypeStruct((NP // 128, 128), jnp.float32)
    po3 = po.reshape(NW, NP // 128, 128)
    pi3 = pi.reshape(NW, NP // 128, 128)
    pf3 = pf.reshape(NW, NP // 128, 128)
    no, ni, fl = pl.pallas_call(_norm_body, out_shape=(o, o, o))(po3, pi3, pf3)
    return no.reshape(NP), ni.reshape(NP), fl.reshape(NP)


def _emb_body(z_hbm, tab_hbm, no_hbm, y0_hbm, zb, nb, rows, sem):
    c = lax.axis_index("c")
    s = lax.axis_index("s")
    wid = c * NS + s
    base = wid * (NP // NW)
    pltpu.sync_copy(z_hbm.at[pl.ds(base, NP // NW)], zb)
    pltpu.sync_copy(no_hbm.at[pl.ds(base, NP // NW)], nb)
    pltpu.async_copy(tab_hbm.at[zb], rows, sem).wait()

    def scale(g, _):
        nv16 = nb[pl.ds(g * 16, 16)]
        for r16 in range(16):
            r = g * 16 + r16
            nv = nv16[r16]
            for cc in range(8):
                rows[r, pl.ds(cc * 16, 16)] = rows[r, pl.ds(cc * 16, 16)] * nv
        return _

    lax.fori_loop(0, NP // NW // 16, scale, 0)
    pltpu.sync_copy(rows, y0_hbm.at[pl.ds(base, NP // NW)])


def _k3(zpad, z_table, norm_out):
    return pl.kernel(
        _emb_body,
        out_type=jax.ShapeDtypeStruct((NP, H), jnp.float32),
        mesh=_MESH,
        compiler_params=_PARAMS,
        scratch_types=[
            pltpu.VMEM((NP // NW,), jnp.int32),
            pltpu.VMEM((NP // NW,), jnp.float32),
            pltpu.VMEM((NP // NW, H), jnp.float32),
            pltpu.SemaphoreType.DMA,
        ],
    )(zpad, z_table, norm_out)


def _layer_body(pred_mode, relu, scale_out, *refs):
    if pred_mode == "flag":
        (y_hbm, src_hbm, dst_hbm, ni_hbm, no_hbm, bias_hbm, flg_hbm, y_out,
         sbuf, dbuf, stg_s, stg_d, idx_s0, idx_d0, idx_s1, idx_d1,
         rows0, rows1, nrows,
         ninb, noutb, biasb, flgb, agg,
         semg0, semg1, sems0, sems1) = refs
    elif pred_mode == "pair":
        (y_hbm, src_hbm, dst_hbm, ni_hbm, no_hbm, bias_hbm, pair_hbm, y_out,
         sbuf, dbuf, stg_s, stg_d, idx_s0, idx_d0, idx_s1, idx_d1,
         rows0, rows1, nrows,
         ninb, noutb, biasb, pairb, agg,
         semg0, semg1, sems0, sems1) = refs
    else:
        (y_hbm, src_hbm, dst_hbm, ni_hbm, no_hbm, bias_hbm, y_out,
         sbuf, dbuf, stg_s, stg_d, idx_s0, idx_d0, idx_s1, idx_d1,
         rows0, rows1, nrows,
         ninb, noutb, biasb, agg,
         semg0, semg1, sems0, sems1) = refs

    c = lax.axis_index("c")
    s = lax.axis_index("s")
    clo = c * NH

    nbase = clo + s * NT
    pltpu.sync_copy(ni_hbm.at[pl.ds(nbase, NT)], ninb)
    pltpu.sync_copy(no_hbm.at[pl.ds(nbase, NT)], noutb)
    pltpu.sync_copy(bias_hbm, biasb)
    if pred_mode == "flag":
        pltpu.sync_copy(flg_hbm, flgb)
    if pred_mode == "pair":
        pltpu.sync_copy(pair_hbm, pairb)
        pairv = pairb[pl.ds(0, 16)]
        u = pairv[0]
        v = pairv[1]

    def zfill(gq, _):
        for r16 in range(16):
            for cc in range(8):
                nrows[gq * 16 + r16, pl.ds(cc * 16, 16)] = (
                    jnp.zeros((16,), jnp.float32))
        return _

    lax.fori_loop(0, 5, zfill, 0)
    zspan = (AGG_ROWS + NS - 1) // NS

    def zloop(k, _):
        zb = jnp.minimum(s * zspan + k * 80, AGG_ROWS - 80)
        pltpu.sync_copy(nrows, agg.at[pl.ds(zb, 80)])
        return _

    lax.fori_loop(0, (zspan + 79) // 80, zloop, 0)
    plsc.subcore_barrier()

    ebase = s * EPT

    def chunk(k, m_cnt):
        pltpu.sync_copy(src_hbm.at[pl.ds(ebase + k * SCAN, SCAN)], sbuf)
        pltpu.sync_copy(dst_hbm.at[pl.ds(ebase + k * SCAN, SCAN)], dbuf)

        def grp16(o, dvq, m_cnt):
            m = (dvq >= clo) & (dvq < clo + NH)
            if pred_mode == "flag":
                fv = plsc.load_gather(flgb, [dvq])
                m = m & (fv > 0.0)
            elif pred_mode == "pair":
                m = m & ((dvq == u) | (dvq == v))
            cnt = plsc.all_reduce_population_count(m)[0]
            sv = sbuf[pl.ds(o, 16)]
            pos = m_cnt + plsc.cumsum(_mask_to_i32(m)) - 1
            plsc.store_scatter(stg_s, [pos], sv, mask=m)
            plsc.store_scatter(stg_d, [pos], dvq - clo, mask=m)
            return m_cnt + cnt

        def grp(g, m_cnt):
            for q in range(4):
                o = g * 64 + q * 16
                m_cnt = grp16(o, dbuf[pl.ds(o, 16)], m_cnt)
            return m_cnt

        m_cnt = lax.fori_loop(0, SCAN // 64, grp, m_cnt)
        for o in range((SCAN // 64) * 64, SCAN, 16):
            m_cnt = grp16(o, dbuf[pl.ds(o, 16)], m_cnt)
        return m_cnt

    m_fin = lax.fori_loop(0, EPT // SCAN, chunk, jnp.int32(0))

    d_pad = (CH - lax.rem(m_fin, CH)) % CH
    iota = lax.iota(jnp.int32, 16)
    for t in range(CH // 16):
        off = t * 16 + iota
        mp = off < d_pad
        plsc.store_scatter(stg_s, [m_fin + off], _zero_i(), mask=mp)
        plsc.store_scatter(stg_d, [m_fin + off],
                           jnp.full((16,), DUMMY, jnp.int32), mask=mp)

    def proc(j, _):
        a64 = j * CH
        for t in range(4):
            idx_s0[pl.ds(t * 16, 16)] = stg_s[pl.ds(a64 + t * 16, 16)]
            idx_d0[pl.ds(t * 16, 16)] = stg_d[pl.ds(a64 + t * 16, 16)]
            idx_s1[pl.ds(t * 16, 16)] = stg_s[pl.ds(a64 + 64 + t * 16, 16)]
            idx_d1[pl.ds(t * 16, 16)] = stg_d[pl.ds(a64 + 64 + t * 16, 16)]
        h0 = pltpu.async_copy(y_hbm.at[idx_s0], rows0, semg0)
        h1 = pltpu.async_copy(y_hbm.at[idx_s1], rows1, semg1)
        h0.wait()
        hs0 = pltpu.async_copy(rows0, agg.at[idx_d0], sems0, add=True)
        h1.wait()
        hs1 = pltpu.async_copy(rows1, agg.at[idx_d1], sems1, add=True)
        hs0.wait()
        hs1.wait()
        return _

    lax.fori_loop(0, (m_fin + d_pad) // CH, proc, 0)
    plsc.subcore_barrier()

    for half in range(NT // 80):
        rlo = s * NT + half * 80

        def node(g, _):
            ni16 = ninb[pl.ds(half * 80 + g * 16, 16)]
            no16 = noutb[pl.ds(half * 80 + g * 16, 16)]
            for r16 in range(16):
                r = g * 16 + r16
                ni = ni16[r16]
                no = no16[r16]
                for cc in range(8):
                    val = nrows[r, pl.ds(cc * 16, 16)]
                    val = val * ni + biasb[pl.ds(cc * 16, 16)]
                    if relu:
                        val = jnp.maximum(val, 0.0)
                    if scale_out:
                        val = val * no
                    nrows[r, pl.ds(cc * 16, 16)] = val
            return _

        pltpu.sync_copy(agg.at[pl.ds(rlo, 80)], nrows)
        lax.fori_loop(0, 5, node, 0)
        pltpu.sync_copy(nrows,
                        y_out.at[pl.ds(nbase + half * 80, 80)])


def _k4(pred_mode, relu, scale_out, y, src, dst, norm_in, norm_out, bias,
        flag=None, pair16=None):
    scratch = [
        pltpu.VMEM((SCAN,), jnp.int32),
        pltpu.VMEM((SCAN,), jnp.int32),
        pltpu.VMEM((CAP,), jnp.int32),
        pltpu.VMEM((CAP,), jnp.int32),
        pltpu.VMEM((64,), jnp.int32),
        pltpu.VMEM((64,), jnp.int32),
        pltpu.VMEM((64,), jnp.int32),
        pltpu.VMEM((64,), jnp.int32),
        pltpu.VMEM((64, H), jnp.float32),
        pltpu.VMEM((64, H), jnp.float32),
        pltpu.VMEM((80, H), jnp.float32),
        pltpu.VMEM((NT,), jnp.float32),
        pltpu.VMEM((NT,), jnp.float32),
        pltpu.VMEM((H,), jnp.float32),
    ]
    args = [y, src, dst, norm_in, norm_out, bias]
    if pred_mode == "flag":
        scratch.append(pltpu.VMEM((NP,), jnp.float32))
        args.append(flag)
    elif pred_mode == "pair":
        scratch.append(pltpu.VMEM((16,), jnp.int32))
        args.append(pair16)
    scratch += [
        pltpu.VMEM_SHARED((AGG_ROWS, H), jnp.float32),
        pltpu.SemaphoreType.DMA,
        pltpu.SemaphoreType.DMA,
        pltpu.SemaphoreType.DMA,
        pltpu.SemaphoreType.DMA,
    ]
    return pl.kernel(
        functools.partial(_layer_body, pred_mode, relu, scale_out),
        out_type=jax.ShapeDtypeStruct((NP, H), jnp.float32),
        mesh=_MESH,
        compiler_params=_PARAMS,
        scratch_types=scratch,
    )(*args)


def _mlp_body(x3_hbm, pair_hbm, w1_hbm, b1_hbm, w2_hbm, b2_hbm, out_hbm,
              pairb, rows, w1b, b1b, w2b, b2b, outb, sem):
    c = lax.axis_index("c")
    s = lax.axis_index("s")

    @pl.when((c == 0) & (s == 0))
    def _():
        pltpu.sync_copy(pair_hbm, pairb)
        pltpu.async_copy(x3_hbm.at[pairb], rows, sem).wait()
        pltpu.sync_copy(w1_hbm, w1b)
        pltpu.sync_copy(b1_hbm, b1b)
        pltpu.sync_copy(w2_hbm, w2b)
        pltpu.sync_copy(b2_hbm, b2b)
        for cc in range(8):
            sl = pl.ds(cc * 16, 16)
            rows[0, sl] = rows[0, sl] * rows[1, sl]

        def fma(g, accs):
            pv16 = rows[0, pl.ds(g * 16, 16)]
            for i16 in range(16):
                i = g * 16 + i16
                pv = pv16[i16]
                accs = tuple(
                    accs[cc] + pv * w1b[i, pl.ds(cc * 16, 16)]
                    for cc in range(8)
                )
            return accs

        accs = lax.fori_loop(
            0, H // 16, fma,
            tuple(jnp.zeros((16,), jnp.float32) for _ in range(8))
        )
        b2v = b2b[pl.ds(0, 16)]
        tot = b2v[0]
        for cc in range(8):
            sl = pl.ds(cc * 16, 16)
            h = jnp.maximum(accs[cc] + b1b[sl], 0.0)
            tot = tot + jnp.sum(h * w2b[sl])
        outb[pl.ds(0, 16)] = jnp.where(lax.iota(jnp.int32, 16) == 0, tot, 0.0)
        pltpu.sync_copy(outb, out_hbm)


def _k5(x3, pair16, W1, b1, W2r, b2pad):
    return pl.kernel(
        _mlp_body,
        out_type=jax.ShapeDtypeStruct((16,), jnp.float32),
        mesh=_MESH,
        compiler_params=_PARAMS,
        scratch_types=[
            pltpu.VMEM((16,), jnp.int32),
            pltpu.VMEM((16, H), jnp.float32),
            pltpu.VMEM((H, H), jnp.float32),
            pltpu.VMEM((H,), jnp.float32),
            pltpu.VMEM((H,), jnp.float32),
            pltpu.VMEM((16,), jnp.float32),
            pltpu.VMEM((16,), jnp.float32),
            pltpu.SemaphoreType.DMA,
        ],
    )(x3, pair16, W1, b1, W2r, b2pad)


def kernel(g, z, pair_nodes, z_table, bias0, bias1, bias2, W1, b1, W2, b2):
    src = g[0]
    dst = g[1]
    zpad = jnp.pad(z, (0, NP - N))
    pair16 = jnp.pad(pair_nodes, (0, 14))
    b2pad = jnp.pad(b2, (0, 15))
    W2r = W2.reshape(H)

    po, pi, pf = _k1(src, dst, pair16)
    norm_out, norm_in, flag = _k2(po, pi, pf)
    y0 = _k3(zpad, z_table, norm_out)
    y1 = _k4("none", True, True, y0, src, dst, norm_in, norm_out, bias0)
    y2 = _k4("flag", True, True, y1, src, dst, norm_in, norm_out, bias1,
             flag=flag)
    x3 = _k4("pair", False, False, y2, src, dst, norm_in, norm_out, bias2,
             pair16=pair16)
    out16 = _k5(x3, pair16, W1, b1, W2r, b2pad)
    return out16[:1]
